# baseline (speedup 1.0000x reference)
import functools

import numpy as np
import jax
import jax.numpy as jnp
from jax import lax
from jax.experimental import pallas as pl
from jax.experimental.pallas import tpu as pltpu
from jax.experimental.pallas import tpu_sc as plsc

NC, NS = 2, 16
NW = NC * NS
CB = 128
NF = 6
BE = 640
BR = 40


def _sc_gather(n_chunks, dim):
    T = -(-n_chunks // NW)
    mesh = plsc.VectorSubcoreMesh(core_axis_name="c", subcore_axis_name="s")

    @functools.partial(
        pl.kernel,
        out_type=jax.ShapeDtypeStruct((n_chunks * CB, dim), jnp.float32),
        mesh=mesh,
        scratch_types=[
            pltpu.VMEM((CB,), jnp.int32),
            pltpu.VMEM((CB, dim), jnp.float32),
            pltpu.SemaphoreType.DMA,
        ],
    )
    def k(table_hbm, idx_hbm, out_hbm, idx_v, rows_v, sem):
        w = lax.axis_index("s") * NC + lax.axis_index("c")

        @pl.loop(0, T)
        def _chunks(t):
            cid = t * NW + w

            @pl.when(cid < n_chunks)
            def _():
                pltpu.sync_copy(idx_hbm.at[cid], idx_v)
                pltpu.async_copy(table_hbm.at[idx_v], rows_v, sem).wait()
                pltpu.sync_copy(rows_v, out_hbm.at[pl.ds(cid * CB, CB)])

    return k


def _sc_scatter(n_chunks_half, np_rows, na_pad, dim):
    T = -(-n_chunks_half // NS)
    big = -(-na_pad // (NS * 8)) * 8
    tail = na_pad - big * (NS - 1)
    assert tail > 0 and tail % 8 == 0 and na_pad % 8 == 0
    mesh = plsc.VectorSubcoreMesh(core_axis_name="c", subcore_axis_name="s")

    @functools.partial(
        pl.kernel,
        out_type=jax.ShapeDtypeStruct((NC * np_rows, dim), jnp.float32),
        mesh=mesh,
        scratch_types=[
            pltpu.VMEM((CB,), jnp.int32),
            pltpu.VMEM((CB, dim), jnp.float32),
            pltpu.VMEM((max(big, tail), dim), jnp.float32),
            pltpu.VMEM_SHARED((na_pad, dim), jnp.float32),
            pltpu.SemaphoreType.DMA,
        ],
    )
    def k(vals_hbm, idx_hbm, out_hbm, i_v, v_v, stage_v, acc_sh, sem):
        c = lax.axis_index("c")
        s = lax.axis_index("s")

        @pl.loop(0, max(big, tail))
        def _zero(i):
            for j in range(dim // 16):
                stage_v[i, pl.ds(j * 16, 16)] = jnp.zeros((16,), jnp.float32)

        @pl.when(s < NS - 1)
        def _():
            pltpu.sync_copy(stage_v.at[pl.ds(0, big)],
                            acc_sh.at[pl.ds(s * big, big)])

        @pl.when(s == NS - 1)
        def _():
            pltpu.sync_copy(stage_v.at[pl.ds(0, tail)],
                            acc_sh.at[pl.ds((NS - 1) * big, tail)])

        plsc.subcore_barrier()

        @pl.loop(0, T)
        def _chunks(t):
            cid = t * NS + s

            @pl.when(cid < n_chunks_half)
            def _():
                gcid = c * n_chunks_half + cid
                pltpu.sync_copy(idx_hbm.at[gcid], i_v)
                pltpu.sync_copy(vals_hbm.at[pl.ds(gcid * CB, CB)], v_v)
                pltpu.sync_copy(v_v, acc_sh.at[i_v], add=True)

        plsc.subcore_barrier()
        base = (1 - c) * np_rows

        @pl.when(s < NS - 1)
        def _():
            pltpu.sync_copy(acc_sh.at[pl.ds(s * big, big)],
                            stage_v.at[pl.ds(0, big)])
            pltpu.sync_copy(stage_v.at[pl.ds(0, big)],
                            out_hbm.at[pl.ds(base + s * big, big)])

        @pl.when(s == NS - 1)
        def _():
            pltpu.sync_copy(acc_sh.at[pl.ds((NS - 1) * big, tail)],
                            stage_v.at[pl.ds(0, tail)])
            pltpu.sync_copy(
                stage_v.at[pl.ds(0, tail)],
                out_hbm.at[pl.ds(base + (NS - 1) * big, tail)])

    return k


def _tc_embed(np2, dim):
    def body(z_r, emb_r, y_r):
        cols = lax.broadcasted_iota(jnp.int32, (BE, dim), 1)
        oh = (z_r[...] == cols).astype(jnp.float32)
        y_r[...] = jnp.dot(oh, emb_r[...], preferred_element_type=jnp.float32)

    return pl.pallas_call(
        body,
        grid=(np2 // BE,),
        in_specs=[pl.BlockSpec((BE, 1), lambda i: (i, 0)),
                  pl.BlockSpec((dim, dim), lambda i: (0, 0))],
        out_specs=pl.BlockSpec((BE, dim), lambda i: (i, 0)),
        out_shape=jax.ShapeDtypeStruct((np2, dim), jnp.float32),
    )


def _tc_tp(e_rows, dim):
    mu = np.linspace(0.0, 8.0, 5)
    half_blocks = e_rows // (2 * BE)

    def body(rows_ref, r_ref, w_ref, out_ref):
        rows = rows_ref[...]
        rr = r_ref[...]
        z = [rows * jnp.exp(-0.125 * (rr - mu[i]) ** 2) for i in range(5)]
        z.append(rows)
        zc = jnp.concatenate(z, axis=1)
        s = jnp.dot(zc, w_ref[0], preferred_element_type=jnp.float32)
        out_ref[...] = s * jax.nn.sigmoid(s)

    return pl.pallas_call(
        body,
        grid=(e_rows // BE,),
        in_specs=[
            pl.BlockSpec((BE, dim), lambda i: (i, 0)),
            pl.BlockSpec((BE, 1), lambda i: (i, 0)),
            pl.BlockSpec((1, NF * dim, dim),
                         lambda i: (i // half_blocks, 0, 0)),
        ],
        out_specs=pl.BlockSpec((BE, dim), lambda i: (i, 0)),
        out_shape=jax.ShapeDtypeStruct((e_rows, dim), jnp.float32),
    )


def _tc_update(np2, dim):
    def body(y_r, d_r, o_r):
        o_r[...] = y_r[...] + d_r[...]

    bs = pl.BlockSpec((BE, dim), lambda i: (i, 0))
    return pl.pallas_call(
        body,
        grid=(np2 // BE,),
        in_specs=[bs, bs],
        out_specs=bs,
        out_shape=jax.ShapeDtypeStruct((np2, dim), jnp.float32),
    )


def _tc_readout(na, np_rows, dim):
    hb = na // BR
    skip = (np_rows - na) // BR

    def imap(i):
        return (jnp.where(i < hb, i, i + skip), 0)

    def body(y_r, d_r, wro_r, bro_r, out_ref):
        t = y_r[...] + d_r[...]
        v = jnp.dot(t, wro_r[...],
                    preferred_element_type=jnp.float32) + bro_r[0, 0]
        ps = jnp.sum(v * jax.nn.sigmoid(v))

        @pl.when(pl.program_id(0) == 0)
        def _():
            out_ref[0, 0] = 0.0

        out_ref[0, 0] += ps

    bs = pl.BlockSpec((BR, dim), imap)
    return pl.pallas_call(
        body,
        grid=(2 * hb,),
        in_specs=[bs, bs,
                  pl.BlockSpec((dim, 1), lambda i: (0, 0)),
                  pl.BlockSpec(memory_space=pltpu.SMEM)],
        out_specs=pl.BlockSpec(memory_space=pltpu.SMEM),
        out_shape=jax.ShapeDtypeStruct((1, 1), jnp.float32),
    )


def kernel(z0, z1, src, dst, r, r_hat, edges, natoms0, natoms1,
           W_emb, b_emb, Ws2d, Wd2s, W_ro, b_ro):
    E = src.shape[0]
    dim = W_emb.shape[1]
    na0, na1 = z0.shape[0], z1.shape[0]
    n_layers = Ws2d.shape[0]
    np_rows = -(-max(na0, na1) // CB) * CB
    na_pad = -(-max(na0, na1) // NS) * NS

    i32 = jnp.int32
    srcc = src.astype(i32).reshape(E // CB, CB)
    dstc = dst.astype(i32).reshape(E // CB, CB)
    gidx = jnp.concatenate([srcc, dstc + np_rows], axis=0)
    sidx = jnp.concatenate([dstc, srcc], axis=0)
    r_col = r.astype(jnp.float32).reshape(E, 1)
    r2 = jnp.concatenate([r_col, r_col], axis=0)
    emb = W_emb.astype(jnp.float32) + b_emb[None, :].astype(jnp.float32)
    emb_pad = jnp.zeros((dim, dim), jnp.float32).at[:emb.shape[0]].set(emb)
    z_cat = jnp.concatenate(
        [z0.astype(i32), jnp.zeros((np_rows - na0,), i32),
         z1.astype(i32), jnp.zeros((np_rows - na1,), i32)]).reshape(-1, 1)

    scale = (1.0 / np.sqrt(NF * dim)) / jnp.sqrt(
        jnp.float32(natoms0 + natoms1))

    g = _sc_gather(2 * E // CB, dim)
    sc = _sc_scatter(E // CB, np_rows, na_pad, dim)
    tp = _tc_tp(2 * E, dim)
    upd = _tc_update(2 * np_rows, dim)

    y = _tc_embed(2 * np_rows, dim)(z_cat, emb_pad)

    for l in range(n_layers):
        w_cat = jnp.stack([Ws2d[l].reshape(NF * dim, dim),
                           Wd2s[l].reshape(NF * dim, dim)]) * scale
        rows = g(y, gidx)
        msgs = tp(rows, r2, w_cat)
        delta = sc(msgs, sidx)
        if l < n_layers - 1:
            y = upd(y, delta)

    out = _tc_readout(na0, np_rows, dim)(
        y, delta,
        W_ro.astype(jnp.float32), b_ro.reshape(1, 1).astype(jnp.float32))
    return out.reshape(())


# speedup vs baseline: 1.0732x; 1.0732x over previous
import functools

import numpy as np
import jax
import jax.numpy as jnp
from jax import lax
from jax.experimental import pallas as pl
from jax.experimental.pallas import tpu as pltpu
from jax.experimental.pallas import tpu_sc as plsc

NC, NS = 2, 16
NW = NC * NS
CB = 128
NF = 6
BE = 640
BR = 40
GU = 6
SU = 4


def _sc_gather(n_chunks, dim):
    T = -(-n_chunks // NW)
    mesh = plsc.VectorSubcoreMesh(core_axis_name="c", subcore_axis_name="s")

    @functools.partial(
        pl.kernel,
        out_type=jax.ShapeDtypeStruct((n_chunks * CB, dim), jnp.float32),
        mesh=mesh,
        scratch_types=(
            [pltpu.VMEM((CB,), jnp.int32)] * GU
            + [pltpu.VMEM((CB, dim), jnp.float32)] * GU
            + [pltpu.SemaphoreType.DMA] * 3
        ),
    )
    def k(table_hbm, idx_hbm, out_hbm, *scr):
        idx_v = scr[:GU]
        rows_v = scr[GU:2 * GU]
        si, sg, sw = scr[2 * GU:]
        w = lax.axis_index("s") * NC + lax.axis_index("c")

        @pl.loop(0, -(-T // GU))
        def _groups(g):
            dgs = []
            for u in range(GU):
                cid = (g * GU + u) * NW + w

                @pl.when(cid < n_chunks)
                def _(u=u, cid=cid):
                    pltpu.sync_copy(idx_hbm.at[cid], idx_v[u])
                    dgs.append(pltpu.async_copy(
                        table_hbm.at[idx_v[u]], rows_v[u], sg))

            for u in range(GU):
                cid = (g * GU + u) * NW + w

                @pl.when(cid < n_chunks)
                def _(u=u, cid=cid):
                    dgs[u].wait()
                    pltpu.sync_copy(
                        rows_v[u], out_hbm.at[pl.ds(cid * CB, CB)])

    return k


def _sc_scatter(n_chunks_half, np_rows, na_pad, dim):
    T = -(-n_chunks_half // NS)
    big = -(-na_pad // (NS * 8)) * 8
    tail = na_pad - big * (NS - 1)
    assert tail > 0 and tail % 8 == 0 and na_pad % 8 == 0
    mesh = plsc.VectorSubcoreMesh(core_axis_name="c", subcore_axis_name="s")

    @functools.partial(
        pl.kernel,
        out_type=jax.ShapeDtypeStruct((NC * np_rows, dim), jnp.float32),
        mesh=mesh,
        scratch_types=(
            [pltpu.VMEM((CB,), jnp.int32)] * SU
            + [pltpu.VMEM((CB, dim), jnp.float32)] * SU
            + [pltpu.VMEM((max(big, tail), dim), jnp.float32),
               pltpu.VMEM_SHARED((na_pad, dim), jnp.float32)]
            + [pltpu.SemaphoreType.DMA] * 3
        ),
    )
    def k(vals_hbm, idx_hbm, out_hbm, *scr):
        i_v = scr[:SU]
        v_v = scr[SU:2 * SU]
        stage_v, acc_sh = scr[2 * SU:2 * SU + 2]
        si, sv, sa = scr[2 * SU + 2:]
        c = lax.axis_index("c")
        s = lax.axis_index("s")

        @pl.loop(0, max(big, tail))
        def _zero(i):
            for j in range(dim // 16):
                stage_v[i, pl.ds(j * 16, 16)] = jnp.zeros((16,), jnp.float32)

        @pl.when(s < NS - 1)
        def _():
            pltpu.sync_copy(stage_v.at[pl.ds(0, big)],
                            acc_sh.at[pl.ds(s * big, big)])

        @pl.when(s == NS - 1)
        def _():
            pltpu.sync_copy(stage_v.at[pl.ds(0, tail)],
                            acc_sh.at[pl.ds((NS - 1) * big, tail)])

        plsc.subcore_barrier()

        @pl.loop(0, T)
        def _chunks(t):
            cid = t * NS + s

            @pl.when(cid < n_chunks_half)
            def _():
                gcid = c * n_chunks_half + cid
                pltpu.sync_copy(idx_hbm.at[gcid], i_v[0])
                pltpu.sync_copy(vals_hbm.at[pl.ds(gcid * CB, CB)], v_v[0])
                pltpu.sync_copy(v_v[0], acc_sh.at[i_v[0]], add=True)

        plsc.subcore_barrier()
        base = (1 - c) * np_rows

        @pl.when(s < NS - 1)
        def _():
            pltpu.sync_copy(acc_sh.at[pl.ds(s * big, big)],
                            stage_v.at[pl.ds(0, big)])
            pltpu.sync_copy(stage_v.at[pl.ds(0, big)],
                            out_hbm.at[pl.ds(base + s * big, big)])

        @pl.when(s == NS - 1)
        def _():
            pltpu.sync_copy(acc_sh.at[pl.ds((NS - 1) * big, tail)],
                            stage_v.at[pl.ds(0, tail)])
            pltpu.sync_copy(
                stage_v.at[pl.ds(0, tail)],
                out_hbm.at[pl.ds(base + (NS - 1) * big, tail)])

    return k


def _tc_embed(np2, dim):
    def body(z_r, emb_r, y_r):
        cols = lax.broadcasted_iota(jnp.int32, (BE, dim), 1)
        oh = (z_r[...] == cols).astype(jnp.float32)
        y_r[...] = jnp.dot(oh, emb_r[...], preferred_element_type=jnp.float32)

    return pl.pallas_call(
        body,
        grid=(np2 // BE,),
        in_specs=[pl.BlockSpec((BE, 1), lambda i: (i, 0)),
                  pl.BlockSpec((dim, dim), lambda i: (0, 0))],
        out_specs=pl.BlockSpec((BE, dim), lambda i: (i, 0)),
        out_shape=jax.ShapeDtypeStruct((np2, dim), jnp.float32),
    )


def _tc_tp(e_rows, dim):
    mu = np.linspace(0.0, 8.0, 5)
    half_blocks = e_rows // (2 * BE)

    def body(rows_ref, r_ref, w_ref, out_ref):
        rows = rows_ref[...]
        rr = r_ref[...]
        z = [rows * jnp.exp(-0.125 * (rr - mu[i]) ** 2) for i in range(5)]
        z.append(rows)
        zc = jnp.concatenate(z, axis=1).astype(jnp.bfloat16)
        s = jnp.dot(zc, w_ref[0], preferred_element_type=jnp.float32)
        out_ref[...] = s * jax.nn.sigmoid(s)

    return pl.pallas_call(
        body,
        grid=(e_rows // BE,),
        in_specs=[
            pl.BlockSpec((BE, dim), lambda i: (i, 0)),
            pl.BlockSpec((BE, 1), lambda i: (i, 0)),
            pl.BlockSpec((1, NF * dim, dim),
                         lambda i: (i // half_blocks, 0, 0)),
        ],
        out_specs=pl.BlockSpec((BE, dim), lambda i: (i, 0)),
        out_shape=jax.ShapeDtypeStruct((e_rows, dim), jnp.float32),
    )


def _tc_update(np2, dim):
    def body(y_r, d_r, o_r):
        o_r[...] = y_r[...] + d_r[...]

    bs = pl.BlockSpec((BE, dim), lambda i: (i, 0))
    return pl.pallas_call(
        body,
        grid=(np2 // BE,),
        in_specs=[bs, bs],
        out_specs=bs,
        out_shape=jax.ShapeDtypeStruct((np2, dim), jnp.float32),
    )


def _tc_readout(na, np_rows, dim):
    hb = na // BR
    skip = (np_rows - na) // BR

    def imap(i):
        return (jnp.where(i < hb, i, i + skip), 0)

    def body(y_r, d_r, wro_r, bro_r, out_ref):
        t = y_r[...] + d_r[...]
        v = jnp.dot(t, wro_r[...],
                    preferred_element_type=jnp.float32) + bro_r[0, 0]
        ps = jnp.sum(v * jax.nn.sigmoid(v))

        @pl.when(pl.program_id(0) == 0)
        def _():
            out_ref[0, 0] = 0.0

        out_ref[0, 0] += ps

    bs = pl.BlockSpec((BR, dim), imap)
    return pl.pallas_call(
        body,
        grid=(2 * hb,),
        in_specs=[bs, bs,
                  pl.BlockSpec((dim, 1), lambda i: (0, 0)),
                  pl.BlockSpec(memory_space=pltpu.SMEM)],
        out_specs=pl.BlockSpec(memory_space=pltpu.SMEM),
        out_shape=jax.ShapeDtypeStruct((1, 1), jnp.float32),
    )


def kernel(z0, z1, src, dst, r, r_hat, edges, natoms0, natoms1,
           W_emb, b_emb, Ws2d, Wd2s, W_ro, b_ro):
    E = src.shape[0]
    dim = W_emb.shape[1]
    na0, na1 = z0.shape[0], z1.shape[0]
    n_layers = Ws2d.shape[0]
    np_rows = -(-max(na0, na1) // CB) * CB
    na_pad = -(-max(na0, na1) // NS) * NS

    i32 = jnp.int32
    srcc = src.astype(i32).reshape(E // CB, CB)
    dstc = dst.astype(i32).reshape(E // CB, CB)
    gidx = jnp.concatenate([srcc, dstc + np_rows], axis=0)
    sidx = jnp.concatenate([dstc, srcc], axis=0)
    r_col = r.astype(jnp.float32).reshape(E, 1)
    r2 = jnp.concatenate([r_col, r_col], axis=0)
    emb = W_emb.astype(jnp.float32) + b_emb[None, :].astype(jnp.float32)
    emb_pad = jnp.zeros((dim, dim), jnp.float32).at[:emb.shape[0]].set(emb)
    z_cat = jnp.concatenate(
        [z0.astype(i32), jnp.zeros((np_rows - na0,), i32),
         z1.astype(i32), jnp.zeros((np_rows - na1,), i32)]).reshape(-1, 1)

    scale = (1.0 / np.sqrt(NF * dim)) / jnp.sqrt(
        jnp.float32(natoms0 + natoms1))

    g = _sc_gather(2 * E // CB, dim)
    sc = _sc_scatter(E // CB, np_rows, na_pad, dim)
    tp = _tc_tp(2 * E, dim)
    upd = _tc_update(2 * np_rows, dim)

    y = _tc_embed(2 * np_rows, dim)(z_cat, emb_pad)

    for l in range(n_layers):
        w_cat = (jnp.stack([Ws2d[l].reshape(NF * dim, dim),
                            Wd2s[l].reshape(NF * dim, dim)])
                 * scale).astype(jnp.bfloat16)
        rows = g(y, gidx)
        msgs = tp(rows, r2, w_cat)
        delta = sc(msgs, sidx)
        if l < n_layers - 1:
            y = upd(y, delta)

    out = _tc_readout(na0, np_rows, dim)(
        y, delta,
        W_ro.astype(jnp.float32), b_ro.reshape(1, 1).astype(jnp.float32))
    return out.reshape(())


# speedup vs baseline: 1.3482x; 1.2562x over previous
import functools

import numpy as np
import jax
import jax.numpy as jnp
from jax import lax
from jax.experimental import pallas as pl
from jax.experimental.pallas import tpu as pltpu
from jax.experimental.pallas import tpu_sc as plsc

NC, NS = 2, 16
NW = NC * NS
CB = 128
NF = 6
BE = 640
BR = 1000
GU = 6


def _sc_gather(n_chunks, dim):
    T = -(-n_chunks // NW)
    mesh = plsc.VectorSubcoreMesh(core_axis_name="c", subcore_axis_name="s")

    @functools.partial(
        pl.kernel,
        out_type=jax.ShapeDtypeStruct((n_chunks * CB, dim), jnp.float32),
        mesh=mesh,
        scratch_types=(
            [pltpu.VMEM((CB,), jnp.int32)] * GU
            + [pltpu.VMEM((CB, dim), jnp.float32)] * GU
            + [pltpu.SemaphoreType.DMA]
        ),
    )
    def k(table_hbm, idx_hbm, out_hbm, *scr):
        idx_v = scr[:GU]
        rows_v = scr[GU:2 * GU]
        sg = scr[2 * GU]
        w = lax.axis_index("s") * NC + lax.axis_index("c")

        @pl.loop(0, -(-T // GU))
        def _groups(g):
            dgs = []
            for u in range(GU):
                cid = (g * GU + u) * NW + w

                @pl.when(cid < n_chunks)
                def _(u=u, cid=cid):
                    pltpu.sync_copy(idx_hbm.at[cid], idx_v[u])
                    dgs.append(pltpu.async_copy(
                        table_hbm.at[idx_v[u]], rows_v[u], sg))

            for u in range(GU):
                cid = (g * GU + u) * NW + w

                @pl.when(cid < n_chunks)
                def _(u=u, cid=cid):
                    dgs[u].wait()
                    pltpu.sync_copy(
                        rows_v[u], out_hbm.at[pl.ds(cid * CB, CB)])

    return k


def _sc_scatter(n_chunks, np_rows, dim):
    T = -(-n_chunks // NW)
    rpt = np_rows // NS
    mesh = plsc.VectorSubcoreMesh(core_axis_name="c", subcore_axis_name="s")

    @functools.partial(
        pl.kernel,
        out_type=jax.ShapeDtypeStruct((NC * np_rows, dim), jnp.float32),
        mesh=mesh,
        scratch_types=[
            pltpu.VMEM((CB,), jnp.int32),
            pltpu.VMEM((CB, dim), jnp.float32),
            pltpu.VMEM((rpt, dim), jnp.float32),
            pltpu.VMEM_SHARED((np_rows, dim), jnp.float32),
            pltpu.SemaphoreType.DMA,
        ],
    )
    def k(vals_hbm, idx_hbm, out_hbm, idx_v, val_v, stage_v, acc_sh, sem):
        c = lax.axis_index("c")
        s = lax.axis_index("s")
        w = s * NC + c

        @pl.loop(0, rpt)
        def _zero(i):
            for j in range(dim // 16):
                stage_v[i, pl.ds(j * 16, 16)] = jnp.zeros((16,), jnp.float32)

        pltpu.sync_copy(stage_v, acc_sh.at[pl.ds(s * rpt, rpt)])
        plsc.subcore_barrier()

        @pl.loop(0, T)
        def _chunks(t):
            cid = t * NW + w

            @pl.when(cid < n_chunks)
            def _():
                pltpu.sync_copy(idx_hbm.at[cid], idx_v)
                pltpu.sync_copy(vals_hbm.at[pl.ds(cid * CB, CB)], val_v)
                pltpu.sync_copy(val_v, acc_sh.at[idx_v], add=True)

        plsc.subcore_barrier()
        pltpu.sync_copy(acc_sh.at[pl.ds(s * rpt, rpt)], stage_v)
        pltpu.sync_copy(
            stage_v, out_hbm.at[pl.ds(c * np_rows + s * rpt, rpt)])

    return k


def _tc_embed(np_rows, dim):
    def body(z0_r, z1_r, emb_r, y0_r, y1_r):
        cols = lax.broadcasted_iota(jnp.int32, (BE, dim), 1)
        emb = emb_r[...]
        oh0 = (z0_r[...] == cols).astype(jnp.float32)
        y0_r[...] = jnp.dot(oh0, emb, preferred_element_type=jnp.float32)
        oh1 = (z1_r[...] == cols).astype(jnp.float32)
        y1_r[...] = jnp.dot(oh1, emb, preferred_element_type=jnp.float32)

    zs = pl.BlockSpec((BE, 1), lambda i: (i, 0))
    ys = pl.BlockSpec((BE, dim), lambda i: (i, 0))
    return pl.pallas_call(
        body,
        grid=(np_rows // BE,),
        in_specs=[zs, zs, pl.BlockSpec((dim, dim), lambda i: (0, 0))],
        out_specs=[ys, ys],
        out_shape=[jax.ShapeDtypeStruct((np_rows, dim), jnp.float32)] * 2,
    )


def _tc_tp(e_rows, dim):
    mu = np.linspace(0.0, 8.0, 5)

    def body(rows_ref, r_ref, w_ref, out_ref):
        rows = rows_ref[...]
        rr = r_ref[...]
        z = [rows * jnp.exp(-0.125 * (rr - mu[i]) ** 2) for i in range(5)]
        z.append(rows)
        zc = jnp.concatenate(z, axis=1).astype(jnp.bfloat16)
        s = jnp.dot(zc, w_ref[...], preferred_element_type=jnp.float32)
        out_ref[...] = s * jax.nn.sigmoid(s)

    return pl.pallas_call(
        body,
        grid=(e_rows // BE,),
        in_specs=[
            pl.BlockSpec((BE, dim), lambda i: (i, 0)),
            pl.BlockSpec((BE, 1), lambda i: (i, 0)),
            pl.BlockSpec((NF * dim, dim), lambda i: (0, 0)),
        ],
        out_specs=pl.BlockSpec((BE, dim), lambda i: (i, 0)),
        out_shape=jax.ShapeDtypeStruct((e_rows, dim), jnp.float32),
    )


def _tc_update(np_rows, dim):
    def body(y0_r, a0_r, b0_r, y1_r, a1_r, b1_r, o0_r, o1_r):
        o0_r[...] = y0_r[...] + a0_r[...] + b0_r[...]
        o1_r[...] = y1_r[...] + a1_r[...] + b1_r[...]

    bs = pl.BlockSpec((BE, dim), lambda i: (i, 0))
    return pl.pallas_call(
        body,
        grid=(np_rows // BE,),
        in_specs=[bs] * 6,
        out_specs=[bs, bs],
        out_shape=[jax.ShapeDtypeStruct((np_rows, dim), jnp.float32)] * 2,
    )


def _tc_readout(na, dim):
    def body(y0_r, a0_r, b0_r, y1_r, a1_r, b1_r, wro_r, bro_r, out_ref):
        t0 = y0_r[...] + a0_r[...] + b0_r[...]
        t1 = y1_r[...] + a1_r[...] + b1_r[...]
        v = jnp.dot(jnp.concatenate([t0, t1], axis=0), wro_r[...],
                    preferred_element_type=jnp.float32) + bro_r[0, 0]
        ps = jnp.sum(v * jax.nn.sigmoid(v))

        @pl.when(pl.program_id(0) == 0)
        def _():
            out_ref[0, 0] = 0.0

        out_ref[0, 0] += ps

    bs = pl.BlockSpec((BR, dim), lambda i: (i, 0))
    return pl.pallas_call(
        body,
        grid=(na // BR,),
        in_specs=[bs] * 6 + [
            pl.BlockSpec((dim, 1), lambda i: (0, 0)),
            pl.BlockSpec(memory_space=pltpu.SMEM),
        ],
        out_specs=pl.BlockSpec(memory_space=pltpu.SMEM),
        out_shape=jax.ShapeDtypeStruct((1, 1), jnp.float32),
    )


def kernel(z0, z1, src, dst, r, r_hat, edges, natoms0, natoms1,
           W_emb, b_emb, Ws2d, Wd2s, W_ro, b_ro):
    E = src.shape[0]
    dim = W_emb.shape[1]
    na0, na1 = z0.shape[0], z1.shape[0]
    n_layers = Ws2d.shape[0]
    np_rows = -(-max(na0, na1) // CB) * CB

    i32 = jnp.int32
    srcc = src.astype(i32).reshape(E // CB, CB)
    dstc = dst.astype(i32).reshape(E // CB, CB)
    r_col = r.astype(jnp.float32).reshape(E, 1)
    emb = W_emb.astype(jnp.float32) + b_emb[None, :].astype(jnp.float32)
    emb_pad = jnp.zeros((dim, dim), jnp.float32).at[:emb.shape[0]].set(emb)
    z0p = jnp.concatenate(
        [z0.astype(i32), jnp.zeros((np_rows - na0,), i32)]).reshape(-1, 1)
    z1p = jnp.concatenate(
        [z1.astype(i32), jnp.zeros((np_rows - na1,), i32)]).reshape(-1, 1)

    scale = (1.0 / np.sqrt(NF * dim)) / jnp.sqrt(
        jnp.float32(natoms0 + natoms1))

    g = _sc_gather(E // CB, dim)
    scat = _sc_scatter(E // CB, np_rows, dim)
    tp = _tc_tp(E, dim)
    upd = _tc_update(np_rows, dim)

    y0, y1 = _tc_embed(np_rows, dim)(z0p, z1p, emb_pad)

    for l in range(n_layers):
        w_s2d = (Ws2d[l].reshape(NF * dim, dim) * scale).astype(jnp.bfloat16)
        w_d2s = (Wd2s[l].reshape(NF * dim, dim) * scale).astype(jnp.bfloat16)
        rows_s = g(y0, srcc)
        rows_d = g(y1, dstc)
        msg_s2d = tp(rows_s, r_col, w_s2d)
        msg_d2s = tp(rows_d, r_col, w_d2s)
        p1 = scat(msg_s2d, dstc)
        p0 = scat(msg_d2s, srcc)
        if l < n_layers - 1:
            y0, y1 = upd(y0, p0[:np_rows], p0[np_rows:],
                         y1, p1[:np_rows], p1[np_rows:])

    out = _tc_readout(na0, dim)(
        y0, p0[:np_rows], p0[np_rows:], y1, p1[:np_rows], p1[np_rows:],
        W_ro.astype(jnp.float32), b_ro.reshape(1, 1).astype(jnp.float32))
    return out.reshape(())


# speedup vs baseline: 1.3807x; 1.0241x over previous
import functools

import numpy as np
import jax
import jax.numpy as jnp
from jax import lax
from jax.experimental import pallas as pl
from jax.experimental.pallas import tpu as pltpu
from jax.experimental.pallas import tpu_sc as plsc

NC, NS = 2, 16
NW = NC * NS
CB = 128
NF = 6
BE = 640
BR = 1000
GU = 6


def _sc_gather(n_chunks, dim):
    T = -(-n_chunks // NW)
    mesh = plsc.VectorSubcoreMesh(core_axis_name="c", subcore_axis_name="s")

    @functools.partial(
        pl.kernel,
        out_type=jax.ShapeDtypeStruct((n_chunks * CB, dim), jnp.float32),
        mesh=mesh,
        scratch_types=(
            [pltpu.VMEM((CB,), jnp.int32)] * GU
            + [pltpu.VMEM((CB, dim), jnp.float32)] * GU
            + [pltpu.SemaphoreType.DMA]
        ),
    )
    def k(table_hbm, idx_hbm, out_hbm, *scr):
        idx_v = scr[:GU]
        rows_v = scr[GU:2 * GU]
        sg = scr[2 * GU]
        w = lax.axis_index("s") * NC + lax.axis_index("c")

        @pl.loop(0, -(-T // GU))
        def _groups(g):
            dgs = []
            for u in range(GU):
                cid = (g * GU + u) * NW + w

                @pl.when(cid < n_chunks)
                def _(u=u, cid=cid):
                    pltpu.sync_copy(idx_hbm.at[cid], idx_v[u])
                    dgs.append(pltpu.async_copy(
                        table_hbm.at[idx_v[u]], rows_v[u], sg))

            for u in range(GU):
                cid = (g * GU + u) * NW + w

                @pl.when(cid < n_chunks)
                def _(u=u, cid=cid):
                    dgs[u].wait()
                    pltpu.sync_copy(
                        rows_v[u], out_hbm.at[pl.ds(cid * CB, CB)])

    return k


def _sc_scatter(n_chunks, np_rows, dim):
    T = -(-n_chunks // NW)
    rpt = np_rows // NS
    mesh = plsc.VectorSubcoreMesh(core_axis_name="c", subcore_axis_name="s")

    @functools.partial(
        pl.kernel,
        out_type=jax.ShapeDtypeStruct((NC * np_rows, dim), jnp.float32),
        mesh=mesh,
        scratch_types=[
            pltpu.VMEM((CB,), jnp.int32),
            pltpu.VMEM((CB, dim), jnp.float32),
            pltpu.VMEM((rpt, dim), jnp.float32),
            pltpu.VMEM_SHARED((np_rows, dim), jnp.float32),
            pltpu.SemaphoreType.DMA,
        ],
    )
    def k(vals_hbm, idx_hbm, out_hbm, idx_v, val_v, stage_v, acc_sh, sem):
        c = lax.axis_index("c")
        s = lax.axis_index("s")
        w = s * NC + c

        @pl.loop(0, rpt)
        def _zero(i):
            for j in range(dim // 16):
                stage_v[i, pl.ds(j * 16, 16)] = jnp.zeros((16,), jnp.float32)

        pltpu.sync_copy(stage_v, acc_sh.at[pl.ds(s * rpt, rpt)])
        plsc.subcore_barrier()

        @pl.loop(0, T)
        def _chunks(t):
            cid = t * NW + w

            @pl.when(cid < n_chunks)
            def _():
                pltpu.sync_copy(idx_hbm.at[cid], idx_v)
                pltpu.sync_copy(vals_hbm.at[pl.ds(cid * CB, CB)], val_v)
                pltpu.sync_copy(val_v, acc_sh.at[idx_v], add=True)

        plsc.subcore_barrier()
        pltpu.sync_copy(acc_sh.at[pl.ds(s * rpt, rpt)], stage_v)
        pltpu.sync_copy(
            stage_v, out_hbm.at[pl.ds(c * np_rows + s * rpt, rpt)])

    return k


def _tc_embed(np_rows, dim):
    def body(z0_r, z1_r, emb_r, y0_r, y1_r):
        cols = lax.broadcasted_iota(jnp.int32, (BE, dim), 1)
        emb = emb_r[...]
        oh0 = (z0_r[...] == cols).astype(jnp.float32)
        y0_r[...] = jnp.dot(oh0, emb, preferred_element_type=jnp.float32)
        oh1 = (z1_r[...] == cols).astype(jnp.float32)
        y1_r[...] = jnp.dot(oh1, emb, preferred_element_type=jnp.float32)

    zs = pl.BlockSpec((BE, 1), lambda i: (i, 0))
    ys = pl.BlockSpec((BE, dim), lambda i: (i, 0))
    return pl.pallas_call(
        body,
        grid=(np_rows // BE,),
        in_specs=[zs, zs, pl.BlockSpec((dim, dim), lambda i: (0, 0))],
        out_specs=[ys, ys],
        out_shape=[jax.ShapeDtypeStruct((np_rows, dim), jnp.float32)] * 2,
    )


def _tc_tp(e_rows, dim):
    mu = np.linspace(0.0, 8.0, 5)

    def body(rows_ref, r_ref, w_ref, out_ref):
        rows = rows_ref[...].astype(jnp.bfloat16)
        rr = r_ref[...]
        p = jnp.dot(rows, w_ref[...], preferred_element_type=jnp.float32)
        s = p[:, (NF - 1) * dim:]
        for i in range(5):
            ef = jnp.exp(-0.125 * (rr - mu[i]) ** 2)
            s = s + ef * p[:, i * dim:(i + 1) * dim]
        out_ref[...] = s * jax.nn.sigmoid(s)

    return pl.pallas_call(
        body,
        grid=(e_rows // BE,),
        in_specs=[
            pl.BlockSpec((BE, dim), lambda i: (i, 0)),
            pl.BlockSpec((BE, 1), lambda i: (i, 0)),
            pl.BlockSpec((dim, NF * dim), lambda i: (0, 0)),
        ],
        out_specs=pl.BlockSpec((BE, dim), lambda i: (i, 0)),
        out_shape=jax.ShapeDtypeStruct((e_rows, dim), jnp.float32),
    )


def _tc_update(np_rows, dim):
    def body(y0_r, a0_r, b0_r, y1_r, a1_r, b1_r, o0_r, o1_r):
        o0_r[...] = y0_r[...] + a0_r[...] + b0_r[...]
        o1_r[...] = y1_r[...] + a1_r[...] + b1_r[...]

    bs = pl.BlockSpec((BE, dim), lambda i: (i, 0))
    return pl.pallas_call(
        body,
        grid=(np_rows // BE,),
        in_specs=[bs] * 6,
        out_specs=[bs, bs],
        out_shape=[jax.ShapeDtypeStruct((np_rows, dim), jnp.float32)] * 2,
    )


def _tc_readout(na, dim):
    def body(y0_r, a0_r, b0_r, y1_r, a1_r, b1_r, wro_r, bro_r, out_ref):
        t0 = y0_r[...] + a0_r[...] + b0_r[...]
        t1 = y1_r[...] + a1_r[...] + b1_r[...]
        v = jnp.dot(jnp.concatenate([t0, t1], axis=0), wro_r[...],
                    preferred_element_type=jnp.float32) + bro_r[0, 0]
        ps = jnp.sum(v * jax.nn.sigmoid(v))

        @pl.when(pl.program_id(0) == 0)
        def _():
            out_ref[0, 0] = 0.0

        out_ref[0, 0] += ps

    bs = pl.BlockSpec((BR, dim), lambda i: (i, 0))
    return pl.pallas_call(
        body,
        grid=(na // BR,),
        in_specs=[bs] * 6 + [
            pl.BlockSpec((dim, 1), lambda i: (0, 0)),
            pl.BlockSpec(memory_space=pltpu.SMEM),
        ],
        out_specs=pl.BlockSpec(memory_space=pltpu.SMEM),
        out_shape=jax.ShapeDtypeStruct((1, 1), jnp.float32),
    )


def kernel(z0, z1, src, dst, r, r_hat, edges, natoms0, natoms1,
           W_emb, b_emb, Ws2d, Wd2s, W_ro, b_ro):
    E = src.shape[0]
    dim = W_emb.shape[1]
    na0, na1 = z0.shape[0], z1.shape[0]
    n_layers = Ws2d.shape[0]
    np_rows = -(-max(na0, na1) // CB) * CB

    i32 = jnp.int32
    srcc = src.astype(i32).reshape(E // CB, CB)
    dstc = dst.astype(i32).reshape(E // CB, CB)
    r_col = r.astype(jnp.float32).reshape(E, 1)
    emb = W_emb.astype(jnp.float32) + b_emb[None, :].astype(jnp.float32)
    emb_pad = jnp.zeros((dim, dim), jnp.float32).at[:emb.shape[0]].set(emb)
    z0p = jnp.concatenate(
        [z0.astype(i32), jnp.zeros((np_rows - na0,), i32)]).reshape(-1, 1)
    z1p = jnp.concatenate(
        [z1.astype(i32), jnp.zeros((np_rows - na1,), i32)]).reshape(-1, 1)

    scale = (1.0 / np.sqrt(NF * dim)) / jnp.sqrt(
        jnp.float32(natoms0 + natoms1))

    g = _sc_gather(E // CB, dim)
    scat = _sc_scatter(E // CB, np_rows, dim)
    tp = _tc_tp(E, dim)
    upd = _tc_update(np_rows, dim)

    y0, y1 = _tc_embed(np_rows, dim)(z0p, z1p, emb_pad)

    for l in range(n_layers):
        w_s2d = (Ws2d[l].transpose(1, 0, 2).reshape(dim, NF * dim)
                 * scale).astype(jnp.bfloat16)
        w_d2s = (Wd2s[l].transpose(1, 0, 2).reshape(dim, NF * dim)
                 * scale).astype(jnp.bfloat16)
        rows_s = g(y0, srcc)
        rows_d = g(y1, dstc)
        msg_s2d = tp(rows_s, r_col, w_s2d)
        msg_d2s = tp(rows_d, r_col, w_d2s)
        p1 = scat(msg_s2d, dstc)
        p0 = scat(msg_d2s, srcc)
        if l < n_layers - 1:
            y0, y1 = upd(y0, p0[:np_rows], p0[np_rows:],
                         y1, p1[:np_rows], p1[np_rows:])

    out = _tc_readout(na0, dim)(
        y0, p0[:np_rows], p0[np_rows:], y1, p1[:np_rows], p1[np_rows:],
        W_ro.astype(jnp.float32), b_ro.reshape(1, 1).astype(jnp.float32))
    return out.reshape(())


# speedup vs baseline: 1.3810x; 1.0003x over previous
import functools

import numpy as np
import jax
import jax.numpy as jnp
from jax import lax
from jax.experimental import pallas as pl
from jax.experimental.pallas import tpu as pltpu
from jax.experimental.pallas import tpu_sc as plsc

NC, NS = 2, 16
NW = NC * NS
CB = 128
NF = 6
BE = 640
BR = 1000
GU = 6


def _sc_gather(n_chunks, dim, dtype=jnp.float32):
    T = -(-n_chunks // NW)
    mesh = plsc.VectorSubcoreMesh(core_axis_name="c", subcore_axis_name="s")

    @functools.partial(
        pl.kernel,
        out_type=jax.ShapeDtypeStruct((n_chunks * CB, dim), dtype),
        mesh=mesh,
        scratch_types=(
            [pltpu.VMEM((CB,), jnp.int32)] * GU
            + [pltpu.VMEM((CB, dim), dtype)] * GU
            + [pltpu.SemaphoreType.DMA]
        ),
    )
    def k(table_hbm, idx_hbm, out_hbm, *scr):
        idx_v = scr[:GU]
        rows_v = scr[GU:2 * GU]
        sg = scr[2 * GU]
        w = lax.axis_index("s") * NC + lax.axis_index("c")

        @pl.loop(0, -(-T // GU))
        def _groups(g):
            dgs = []
            for u in range(GU):
                cid = (g * GU + u) * NW + w

                @pl.when(cid < n_chunks)
                def _(u=u, cid=cid):
                    pltpu.sync_copy(idx_hbm.at[cid], idx_v[u])
                    dgs.append(pltpu.async_copy(
                        table_hbm.at[idx_v[u]], rows_v[u], sg))

            for u in range(GU):
                cid = (g * GU + u) * NW + w

                @pl.when(cid < n_chunks)
                def _(u=u, cid=cid):
                    dgs[u].wait()
                    pltpu.sync_copy(
                        rows_v[u], out_hbm.at[pl.ds(cid * CB, CB)])

    return k


def _sc_scatter(n_chunks, np_rows, dim):
    T = -(-n_chunks // NW)
    rpt = np_rows // NS
    mesh = plsc.VectorSubcoreMesh(core_axis_name="c", subcore_axis_name="s")

    @functools.partial(
        pl.kernel,
        out_type=jax.ShapeDtypeStruct((NC * np_rows, dim), jnp.float32),
        mesh=mesh,
        scratch_types=[
            pltpu.VMEM((CB,), jnp.int32),
            pltpu.VMEM((CB, dim), jnp.float32),
            pltpu.VMEM((rpt, dim), jnp.float32),
            pltpu.VMEM_SHARED((np_rows, dim), jnp.float32),
            pltpu.SemaphoreType.DMA,
        ],
    )
    def k(vals_hbm, idx_hbm, out_hbm, idx_v, val_v, stage_v, acc_sh, sem):
        c = lax.axis_index("c")
        s = lax.axis_index("s")
        w = s * NC + c

        @pl.loop(0, rpt)
        def _zero(i):
            for j in range(dim // 16):
                stage_v[i, pl.ds(j * 16, 16)] = jnp.zeros((16,), jnp.float32)

        pltpu.sync_copy(stage_v, acc_sh.at[pl.ds(s * rpt, rpt)])
        plsc.subcore_barrier()

        @pl.loop(0, T)
        def _chunks(t):
            cid = t * NW + w

            @pl.when(cid < n_chunks)
            def _():
                pltpu.sync_copy(idx_hbm.at[cid], idx_v)
                pltpu.sync_copy(vals_hbm.at[pl.ds(cid * CB, CB)], val_v)
                pltpu.sync_copy(val_v, acc_sh.at[idx_v], add=True)

        plsc.subcore_barrier()
        pltpu.sync_copy(acc_sh.at[pl.ds(s * rpt, rpt)], stage_v)
        pltpu.sync_copy(
            stage_v, out_hbm.at[pl.ds(c * np_rows + s * rpt, rpt)])

    return k


def _tc_embed(np_rows, dim):
    def body(z0_r, z1_r, emb_r, y0_r, y1_r):
        cols = lax.broadcasted_iota(jnp.int32, (BE, dim), 1)
        emb = emb_r[...]
        oh0 = (z0_r[...] == cols).astype(jnp.float32)
        y0_r[...] = jnp.dot(oh0, emb, preferred_element_type=jnp.float32)
        oh1 = (z1_r[...] == cols).astype(jnp.float32)
        y1_r[...] = jnp.dot(oh1, emb, preferred_element_type=jnp.float32)

    zs = pl.BlockSpec((BE, 1), lambda i: (i, 0))
    ys = pl.BlockSpec((BE, dim), lambda i: (i, 0))
    return pl.pallas_call(
        body,
        grid=(np_rows // BE,),
        in_specs=[zs, zs, pl.BlockSpec((dim, dim), lambda i: (0, 0))],
        out_specs=[ys, ys],
        out_shape=[jax.ShapeDtypeStruct((np_rows, dim), jnp.float32)] * 2,
    )


def _tc_tp(e_rows, dim):
    mu = np.linspace(0.0, 8.0, 5)

    def body(rows_ref, r_ref, w_ref, out_ref):
        rows = rows_ref[...].astype(jnp.bfloat16)
        rr = r_ref[...]
        p = jnp.dot(rows, w_ref[...], preferred_element_type=jnp.float32)
        s = p[:, (NF - 1) * dim:]
        for i in range(5):
            ef = jnp.exp(-0.125 * (rr - mu[i]) ** 2)
            s = s + ef * p[:, i * dim:(i + 1) * dim]
        out_ref[...] = s * jax.nn.sigmoid(s)

    return pl.pallas_call(
        body,
        grid=(e_rows // BE,),
        in_specs=[
            pl.BlockSpec((BE, dim), lambda i: (i, 0)),
            pl.BlockSpec((BE, 1), lambda i: (i, 0)),
            pl.BlockSpec((dim, NF * dim), lambda i: (0, 0)),
        ],
        out_specs=pl.BlockSpec((BE, dim), lambda i: (i, 0)),
        out_shape=jax.ShapeDtypeStruct((e_rows, dim), jnp.float32),
    )


def _tc_update(np_rows, dim):
    def body(y0_r, a0_r, b0_r, y1_r, a1_r, b1_r, o0_r, o1_r):
        o0_r[...] = y0_r[...] + a0_r[...] + b0_r[...]
        o1_r[...] = y1_r[...] + a1_r[...] + b1_r[...]

    bs = pl.BlockSpec((BE, dim), lambda i: (i, 0))
    return pl.pallas_call(
        body,
        grid=(np_rows // BE,),
        in_specs=[bs] * 6,
        out_specs=[bs, bs],
        out_shape=[jax.ShapeDtypeStruct((np_rows, dim), jnp.float32)] * 2,
    )


def _tc_readout(na, dim):
    def body(y0_r, a0_r, b0_r, y1_r, a1_r, b1_r, wro_r, bro_r, out_ref):
        t0 = y0_r[...] + a0_r[...] + b0_r[...]
        t1 = y1_r[...] + a1_r[...] + b1_r[...]
        v = jnp.dot(jnp.concatenate([t0, t1], axis=0), wro_r[...],
                    preferred_element_type=jnp.float32) + bro_r[0, 0]
        ps = jnp.sum(v * jax.nn.sigmoid(v))

        @pl.when(pl.program_id(0) == 0)
        def _():
            out_ref[0, 0] = 0.0

        out_ref[0, 0] += ps

    bs = pl.BlockSpec((BR, dim), lambda i: (i, 0))
    return pl.pallas_call(
        body,
        grid=(na // BR,),
        in_specs=[bs] * 6 + [
            pl.BlockSpec((dim, 1), lambda i: (0, 0)),
            pl.BlockSpec(memory_space=pltpu.SMEM),
        ],
        out_specs=pl.BlockSpec(memory_space=pltpu.SMEM),
        out_shape=jax.ShapeDtypeStruct((1, 1), jnp.float32),
    )


def kernel(z0, z1, src, dst, r, r_hat, edges, natoms0, natoms1,
           W_emb, b_emb, Ws2d, Wd2s, W_ro, b_ro):
    E = src.shape[0]
    dim = W_emb.shape[1]
    na0, na1 = z0.shape[0], z1.shape[0]
    n_layers = Ws2d.shape[0]
    np_rows = -(-max(na0, na1) // CB) * CB

    i32 = jnp.int32
    srcc = src.astype(i32).reshape(E // CB, CB)
    dstc = dst.astype(i32).reshape(E // CB, CB)
    r_col = r.astype(jnp.float32).reshape(E, 1)
    emb = W_emb.astype(jnp.float32) + b_emb[None, :].astype(jnp.float32)
    emb_pad = jnp.zeros((dim, dim), jnp.float32).at[:emb.shape[0]].set(emb)
    z0p = jnp.concatenate(
        [z0.astype(i32), jnp.zeros((np_rows - na0,), i32)]).reshape(-1, 1)
    z1p = jnp.concatenate(
        [z1.astype(i32), jnp.zeros((np_rows - na1,), i32)]).reshape(-1, 1)

    scale = (1.0 / np.sqrt(NF * dim)) / jnp.sqrt(
        jnp.float32(natoms0 + natoms1))

    g = _sc_gather(E // CB, dim)
    scat = _sc_scatter(E // CB, np_rows, dim)
    tp = _tc_tp(E, dim)
    upd = _tc_update(np_rows, dim)

    y0, y1 = _tc_embed(np_rows, dim)(z0p, z1p, emb_pad)

    for l in range(n_layers):
        w_s2d = (Ws2d[l].transpose(1, 0, 2).reshape(dim, NF * dim)
                 * scale).astype(jnp.bfloat16)
        w_d2s = (Wd2s[l].transpose(1, 0, 2).reshape(dim, NF * dim)
                 * scale).astype(jnp.bfloat16)
        rows_s = g(y0, srcc)
        rows_d = g(y1, dstc)
        msg_s2d = tp(rows_s, r_col, w_s2d)
        msg_d2s = tp(rows_d, r_col, w_d2s)
        p1 = scat(msg_s2d, dstc)
        p0 = scat(msg_d2s, srcc)
        if l < n_layers - 1:
            y0, y1 = upd(y0, p0[:np_rows], p0[np_rows:],
                         y1, p1[:np_rows], p1[np_rows:])

    out = _tc_readout(na0, dim)(
        y0, p0[:np_rows], p0[np_rows:], y1, p1[:np_rows], p1[np_rows:],
        W_ro.astype(jnp.float32), b_ro.reshape(1, 1).astype(jnp.float32))
    return out.reshape(())
